# Initial kernel scaffold; baseline (speedup 1.0000x reference)
import jax
import jax.numpy as jnp
from jax.experimental import pallas as pl


def kernel(keys, values, codebook):
    raise NotImplementedError("write your pallas kernel here")


# speedup vs baseline: 1.0790x; 1.0790x over previous
import functools

import jax
import jax.numpy as jnp
from jax.experimental import pallas as pl

HIDDEN = 2048
NUM_CENTROIDS = 1024
SPARSITY_THRESHOLD = 0.1
BLK = 256


def _argmin_body(x_ref, cb_ref, c2_ref, x2_ref, idx_ref):
    xb = x_ref[...]
    m = jax.lax.dot_general(
        xb, cb_ref[...], (((1,), (1,)), ((), ())),
        preferred_element_type=jnp.float32,
    )
    d2 = x2_ref[...] + c2_ref[...] - 2.0 * m
    d = jnp.sqrt(jnp.maximum(d2, 0.0))
    dmin = jnp.min(d, axis=1, keepdims=True)
    ids = jax.lax.broadcasted_iota(jnp.int32, (BLK, NUM_CENTROIDS), 1)
    idx = jnp.min(jnp.where(d == dmin, ids, NUM_CENTROIDS), axis=1)
    idx_ref[...] = idx.reshape(1, 1, BLK)


@functools.partial(jax.jit, static_argnames=("interpret",))
def _argmin_indices(xb, cb_bf16, c2, x2, interpret=False):
    n, h = xb.shape
    k = cb_bf16.shape[0]
    nblk = n // BLK
    idx3 = pl.pallas_call(
        _argmin_body,
        grid=(nblk,),
        in_specs=[
            pl.BlockSpec((BLK, h), lambda i: (i, 0)),
            pl.BlockSpec((k, h), lambda i: (0, 0)),
            pl.BlockSpec((1, k), lambda i: (0, 0)),
            pl.BlockSpec((BLK, 1), lambda i: (i, 0)),
        ],
        out_specs=pl.BlockSpec((1, 1, BLK), lambda i: (i, 0, 0)),
        out_shape=jax.ShapeDtypeStruct((nblk, 1, BLK), jnp.int32),
        interpret=interpret,
    )(xb, cb_bf16, c2, x2)
    return idx3.reshape(n)


def kernel(keys, values, codebook):
    batch, seq, h = keys.shape
    k2d = keys.reshape(-1, h)
    v2d = values.reshape(-1, h)
    cb_bf16 = codebook.astype(jnp.bfloat16)
    c2 = jnp.sum(codebook * codebook, axis=1)[None, :]
    kx2 = jnp.sum(k2d * k2d, axis=1, keepdims=True)
    vx2 = jnp.sum(v2d * v2d, axis=1, keepdims=True)

    key_idx = _argmin_indices(k2d.astype(jnp.bfloat16), cb_bf16, c2, kx2)
    val_idx = _argmin_indices(v2d.astype(jnp.bfloat16), cb_bf16, c2, vx2)

    mask = (jnp.sqrt(c2[0]) > SPARSITY_THRESHOLD).astype(codebook.dtype)
    cb_masked = codebook * mask[:, None]
    keys_c = jnp.take(cb_masked, key_idx, axis=0).reshape(batch, seq, h)
    vals_c = jnp.take(cb_masked, val_idx, axis=0).reshape(batch, seq, h)
    return keys_c, vals_c


# speedup vs baseline: 1.1567x; 1.0720x over previous
import functools

import jax
import jax.numpy as jnp
from jax import lax
from jax.experimental import pallas as pl
from jax.experimental.pallas import tpu as pltpu
from jax.experimental.pallas import tpu_sc as plsc

HIDDEN = 2048
NUM_CENTROIDS = 1024
SPARSITY_THRESHOLD = 0.1
BLK = 256


def _argmin_body(x_ref, cb_ref, c2_ref, x2_ref, idx_ref):
    xb = x_ref[...]
    m = jax.lax.dot_general(
        xb, cb_ref[...], (((1,), (1,)), ((), ())),
        preferred_element_type=jnp.float32,
    )
    d2 = x2_ref[...] + c2_ref[...] - 2.0 * m
    d = jnp.sqrt(jnp.maximum(d2, 0.0))
    dmin = jnp.min(d, axis=1, keepdims=True)
    ids = jax.lax.broadcasted_iota(jnp.int32, (BLK, NUM_CENTROIDS), 1)
    idx = jnp.min(jnp.where(d == dmin, ids, NUM_CENTROIDS), axis=1)
    idx_ref[...] = idx.reshape(1, 1, BLK)


@functools.partial(jax.jit, static_argnames=("interpret",))
def _argmin_indices(xb, cb_bf16, c2, x2, interpret=False):
    n, h = xb.shape
    k = cb_bf16.shape[0]
    nblk = n // BLK
    idx3 = pl.pallas_call(
        _argmin_body,
        grid=(nblk,),
        in_specs=[
            pl.BlockSpec((BLK, h), lambda i: (i, 0)),
            pl.BlockSpec((k, h), lambda i: (0, 0)),
            pl.BlockSpec((1, k), lambda i: (0, 0)),
            pl.BlockSpec((BLK, 1), lambda i: (i, 0)),
        ],
        out_specs=pl.BlockSpec((1, 1, BLK), lambda i: (i, 0, 0)),
        out_shape=jax.ShapeDtypeStruct((nblk, 1, BLK), jnp.int32),
        interpret=interpret,
    )(xb, cb_bf16, c2, x2)
    return idx3.reshape(n)


def _mask_body(cb_ref, out_ref):
    cb = cb_ref[...]
    c2 = jnp.sum(cb * cb, axis=1, keepdims=True)
    msk = (jnp.sqrt(c2) > SPARSITY_THRESHOLD).astype(cb.dtype)
    out_ref[...] = cb * msk


@jax.jit
def _mask_codebook(codebook):
    k, h = codebook.shape
    return pl.pallas_call(
        _mask_body,
        in_specs=[pl.BlockSpec((k, h), lambda: (0, 0))],
        out_specs=pl.BlockSpec((k, h), lambda: (0, 0)),
        out_shape=jax.ShapeDtypeStruct((k, h), jnp.float32),
    )(codebook)


_SC_INFO = plsc.get_sparse_core_info()
_NC = _SC_INFO.num_cores
_NS = _SC_INFO.num_subcores
_NW = _NC * _NS
N_TOK = 16384
B_PER_W = N_TOK // _NW
ROWS = 16
NCHUNK = B_PER_W // ROWS


@functools.partial(
    pl.kernel,
    mesh=plsc.VectorSubcoreMesh(core_axis_name="c", subcore_axis_name="s"),
    out_type=jax.ShapeDtypeStruct((N_TOK, HIDDEN), jnp.float32),
    scratch_types=[
        pltpu.VMEM((B_PER_W,), jnp.int32),
        pltpu.VMEM((ROWS, HIDDEN), jnp.float32),
        pltpu.VMEM((ROWS, HIDDEN), jnp.float32),
        pltpu.SemaphoreType.DMA,
        pltpu.SemaphoreType.DMA,
    ],
)
def _sc_gather(table_hbm, idx_hbm, out_hbm, idx_v, buf0, buf1, sem0, sem1):
    wid = lax.axis_index("s") * _NC + lax.axis_index("c")
    base = wid * B_PER_W
    pltpu.sync_copy(idx_hbm.at[pl.ds(base, B_PER_W)], idx_v)

    def body(c, _):
        h0 = pltpu.async_copy(
            table_hbm.at[idx_v.at[pl.ds(c * ROWS, ROWS)]], buf0, sem0)
        h1 = pltpu.async_copy(
            table_hbm.at[idx_v.at[pl.ds((c + 1) * ROWS, ROWS)]], buf1, sem1)
        h0.wait()
        pltpu.sync_copy(buf0, out_hbm.at[pl.ds(base + c * ROWS, ROWS)])
        h1.wait()
        pltpu.sync_copy(buf1, out_hbm.at[pl.ds(base + (c + 1) * ROWS, ROWS)])
        return ()

    lax.fori_loop(0, NCHUNK // 2, lambda i, _: body(i * 2, ()), (), unroll=False)


def kernel(keys, values, codebook):
    batch, seq, h = keys.shape
    k2d = keys.reshape(-1, h)
    v2d = values.reshape(-1, h)
    cb_bf16 = codebook.astype(jnp.bfloat16)
    c2 = jnp.sum(codebook * codebook, axis=1)[None, :]
    kx2 = jnp.sum(k2d * k2d, axis=1, keepdims=True)
    vx2 = jnp.sum(v2d * v2d, axis=1, keepdims=True)

    key_idx = _argmin_indices(k2d.astype(jnp.bfloat16), cb_bf16, c2, kx2)
    val_idx = _argmin_indices(v2d.astype(jnp.bfloat16), cb_bf16, c2, vx2)

    cb_masked = _mask_codebook(codebook)
    keys_c = _sc_gather(cb_masked, key_idx).reshape(batch, seq, h)
    vals_c = _sc_gather(cb_masked, val_idx).reshape(batch, seq, h)
    return keys_c, vals_c


# speedup vs baseline: 1.1632x; 1.0056x over previous
import functools

import jax
import jax.numpy as jnp
from jax import lax
from jax.experimental import pallas as pl
from jax.experimental.pallas import tpu as pltpu
from jax.experimental.pallas import tpu_sc as plsc

HIDDEN = 2048
NUM_CENTROIDS = 1024
SPARSITY_THRESHOLD = 0.1
BLK = 256


def _argmin_body(x_ref, cb_ref, c2_ref, x2_ref, idx_ref):
    xb = x_ref[...]
    m = jax.lax.dot_general(
        xb, cb_ref[...], (((1,), (1,)), ((), ())),
        preferred_element_type=jnp.float32,
    )
    d2 = x2_ref[...] + c2_ref[...] - 2.0 * m
    d = jnp.sqrt(jnp.maximum(d2, 0.0))
    dmin = jnp.min(d, axis=1, keepdims=True)
    ids = jax.lax.broadcasted_iota(jnp.int32, (BLK, NUM_CENTROIDS), 1)
    idx = jnp.min(jnp.where(d == dmin, ids, NUM_CENTROIDS), axis=1)
    idx_ref[...] = idx.reshape(1, 1, BLK)


@functools.partial(jax.jit, static_argnames=("interpret",))
def _argmin_indices(xb, cb_bf16, c2, x2, interpret=False):
    n, h = xb.shape
    k = cb_bf16.shape[0]
    nblk = n // BLK
    idx3 = pl.pallas_call(
        _argmin_body,
        grid=(nblk,),
        in_specs=[
            pl.BlockSpec((BLK, h), lambda i: (i, 0)),
            pl.BlockSpec((k, h), lambda i: (0, 0)),
            pl.BlockSpec((1, k), lambda i: (0, 0)),
            pl.BlockSpec((BLK, 1), lambda i: (i, 0)),
        ],
        out_specs=pl.BlockSpec((1, 1, BLK), lambda i: (i, 0, 0)),
        out_shape=jax.ShapeDtypeStruct((nblk, 1, BLK), jnp.int32),
        interpret=interpret,
    )(xb, cb_bf16, c2, x2)
    return idx3.reshape(n)


def _mask_body(cb_ref, out_ref):
    cb = cb_ref[...]
    c2 = jnp.sum(cb * cb, axis=1, keepdims=True)
    msk = (jnp.sqrt(c2) > SPARSITY_THRESHOLD).astype(cb.dtype)
    out_ref[...] = cb * msk


@jax.jit
def _mask_codebook(codebook):
    k, h = codebook.shape
    return pl.pallas_call(
        _mask_body,
        in_specs=[pl.BlockSpec((k, h), lambda: (0, 0))],
        out_specs=pl.BlockSpec((k, h), lambda: (0, 0)),
        out_shape=jax.ShapeDtypeStruct((k, h), jnp.float32),
    )(codebook)


_SC_INFO = plsc.get_sparse_core_info()
_NC = _SC_INFO.num_cores
_NS = _SC_INFO.num_subcores
_NW = _NC * _NS
N_TOK = 16384
B_PER_W = N_TOK // _NW
ROWS = 16
NCHUNK = B_PER_W // ROWS


@functools.partial(
    pl.kernel,
    mesh=plsc.VectorSubcoreMesh(core_axis_name="c", subcore_axis_name="s"),
    out_type=jax.ShapeDtypeStruct((N_TOK, HIDDEN), jnp.float32),
    scratch_types=[
        pltpu.VMEM((B_PER_W,), jnp.int32),
        pltpu.VMEM((ROWS, HIDDEN), jnp.float32),
        pltpu.VMEM((ROWS, HIDDEN), jnp.float32),
        pltpu.SemaphoreType.DMA,
        pltpu.SemaphoreType.DMA,
    ],
)
def _sc_gather(table_hbm, idx_hbm, out_hbm, idx_v, buf0, buf1, sem0, sem1):
    wid = lax.axis_index("s") * _NC + lax.axis_index("c")
    base = wid * B_PER_W
    pltpu.sync_copy(idx_hbm.at[pl.ds(base, B_PER_W)], idx_v)
    bufs = (buf0, buf1)
    sems = (sem0, sem1)

    def start_gather(c, b):
        return pltpu.async_copy(
            table_hbm.at[idx_v.at[pl.ds(c * ROWS, ROWS)]], bufs[b], sems[b])

    start_gather(0, 0)
    start_gather(1, 1)

    def outer(i, _):
        for b in range(2):
            c = i * 2 + b
            pltpu.make_async_copy(
                table_hbm.at[idx_v.at[pl.ds(c * ROWS, ROWS)]],
                bufs[b], sems[b]).wait()
            pltpu.sync_copy(bufs[b], out_hbm.at[pl.ds(base + c * ROWS, ROWS)])

            @pl.when(c + 2 < NCHUNK)
            def _():
                start_gather(c + 2, b)
        return ()

    lax.fori_loop(0, NCHUNK // 2, outer, (), unroll=False)


def kernel(keys, values, codebook):
    batch, seq, h = keys.shape
    k2d = keys.reshape(-1, h)
    v2d = values.reshape(-1, h)
    cb_bf16 = codebook.astype(jnp.bfloat16)
    c2 = jnp.sum(codebook * codebook, axis=1)[None, :]
    kx2 = jnp.sum(k2d * k2d, axis=1, keepdims=True)
    vx2 = jnp.sum(v2d * v2d, axis=1, keepdims=True)

    key_idx = _argmin_indices(k2d.astype(jnp.bfloat16), cb_bf16, c2, kx2)
    val_idx = _argmin_indices(v2d.astype(jnp.bfloat16), cb_bf16, c2, vx2)

    cb_masked = _mask_codebook(codebook)
    keys_c = _sc_gather(cb_masked, key_idx).reshape(batch, seq, h)
    vals_c = _sc_gather(cb_masked, val_idx).reshape(batch, seq, h)
    return keys_c, vals_c


# speedup vs baseline: 1.1646x; 1.0013x over previous
import functools

import jax
import jax.numpy as jnp
from jax import lax
from jax.experimental import pallas as pl
from jax.experimental.pallas import tpu as pltpu
from jax.experimental.pallas import tpu_sc as plsc

HIDDEN = 2048
NUM_CENTROIDS = 1024
SPARSITY_THRESHOLD = 0.1
BLK = 256


def _argmin_body(x_ref, cb_ref, c2_ref, x2_ref, idx_ref):
    xb = x_ref[...]
    m = jax.lax.dot_general(
        xb, cb_ref[...], (((1,), (1,)), ((), ())),
        preferred_element_type=jnp.float32,
    )
    d2 = x2_ref[...] + c2_ref[...] - 2.0 * m
    d = jnp.sqrt(jnp.maximum(d2, 0.0))
    dmin = jnp.min(d, axis=1, keepdims=True)
    ids = jax.lax.broadcasted_iota(jnp.int32, (BLK, NUM_CENTROIDS), 1)
    idx = jnp.min(jnp.where(d == dmin, ids, NUM_CENTROIDS), axis=1)
    idx_ref[...] = idx.reshape(1, 1, BLK)


@functools.partial(jax.jit, static_argnames=("interpret",))
def _argmin_indices(xb, cb_bf16, c2, x2, interpret=False):
    n, h = xb.shape
    k = cb_bf16.shape[0]
    nblk = n // BLK
    idx3 = pl.pallas_call(
        _argmin_body,
        grid=(nblk,),
        in_specs=[
            pl.BlockSpec((BLK, h), lambda i: (i, 0)),
            pl.BlockSpec((k, h), lambda i: (0, 0)),
            pl.BlockSpec((1, k), lambda i: (0, 0)),
            pl.BlockSpec((BLK, 1), lambda i: (i, 0)),
        ],
        out_specs=pl.BlockSpec((1, 1, BLK), lambda i: (i, 0, 0)),
        out_shape=jax.ShapeDtypeStruct((nblk, 1, BLK), jnp.int32),
        interpret=interpret,
    )(xb, cb_bf16, c2, x2)
    return idx3.reshape(n)


def _mask_body(cb_ref, out_ref):
    cb = cb_ref[...]
    c2 = jnp.sum(cb * cb, axis=1, keepdims=True)
    msk = (jnp.sqrt(c2) > SPARSITY_THRESHOLD).astype(cb.dtype)
    out_ref[...] = cb * msk


@jax.jit
def _mask_codebook(codebook):
    k, h = codebook.shape
    return pl.pallas_call(
        _mask_body,
        in_specs=[pl.BlockSpec((k, h), lambda: (0, 0))],
        out_specs=pl.BlockSpec((k, h), lambda: (0, 0)),
        out_shape=jax.ShapeDtypeStruct((k, h), jnp.float32),
    )(codebook)


_SC_INFO = plsc.get_sparse_core_info()
_NC = _SC_INFO.num_cores
_NS = _SC_INFO.num_subcores
_NW = _NC * _NS
N_TOK = 16384
B_PER_W = N_TOK // _NW
ROWS = 8
NCHUNK = B_PER_W // ROWS
NBUF = 4


@functools.partial(
    pl.kernel,
    mesh=plsc.VectorSubcoreMesh(core_axis_name="c", subcore_axis_name="s"),
    out_type=jax.ShapeDtypeStruct((N_TOK, HIDDEN), jnp.float32),
    scratch_types=[
        pltpu.VMEM((B_PER_W,), jnp.int32),
        pltpu.VMEM((ROWS, HIDDEN), jnp.float32),
        pltpu.VMEM((ROWS, HIDDEN), jnp.float32),
        pltpu.VMEM((ROWS, HIDDEN), jnp.float32),
        pltpu.VMEM((ROWS, HIDDEN), jnp.float32),
        pltpu.SemaphoreType.DMA,
        pltpu.SemaphoreType.DMA,
        pltpu.SemaphoreType.DMA,
        pltpu.SemaphoreType.DMA,
        pltpu.SemaphoreType.DMA,
        pltpu.SemaphoreType.DMA,
        pltpu.SemaphoreType.DMA,
        pltpu.SemaphoreType.DMA,
    ],
)
def _sc_gather(table_hbm, idx_hbm, out_hbm, idx_v,
               buf0, buf1, buf2, buf3,
               gs0, gs1, gs2, gs3, ws0, ws1, ws2, ws3):
    wid = lax.axis_index("s") * _NC + lax.axis_index("c")
    base = wid * B_PER_W
    pltpu.sync_copy(idx_hbm.at[pl.ds(base, B_PER_W)], idx_v)
    bufs = (buf0, buf1, buf2, buf3)
    gsems = (gs0, gs1, gs2, gs3)
    wsems = (ws0, ws1, ws2, ws3)

    def start_gather(c, b):
        pltpu.async_copy(
            table_hbm.at[idx_v.at[pl.ds(c * ROWS, ROWS)]], bufs[b], gsems[b])

    def wait_gather(c, b):
        pltpu.make_async_copy(
            table_hbm.at[idx_v.at[pl.ds(c * ROWS, ROWS)]],
            bufs[b], gsems[b]).wait()

    def start_write(c, b):
        pltpu.async_copy(
            bufs[b], out_hbm.at[pl.ds(base + c * ROWS, ROWS)], wsems[b])

    def wait_write(c, b):
        pltpu.make_async_copy(
            bufs[b], out_hbm.at[pl.ds(base + c * ROWS, ROWS)], wsems[b]).wait()

    start_gather(0, 0)
    start_gather(1, 1)

    def slot(c, b):
        wait_gather(c, b)
        start_write(c, b)
        bp = (b + 2) % NBUF

        @pl.when(c + 2 < NCHUNK)
        def _():
            @pl.when(c + 2 >= NBUF)
            def _():
                wait_write(c + 2 - NBUF, bp)
            start_gather(c + 2, bp)

    def outer(i, _):
        for b in range(NBUF):
            slot(i * NBUF + b, b)
        return ()

    lax.fori_loop(0, NCHUNK // NBUF, outer, (), unroll=False)
    for w in range(NCHUNK - NBUF, NCHUNK):
        wait_write(w, w % NBUF)


def kernel(keys, values, codebook):
    batch, seq, h = keys.shape
    k2d = keys.reshape(-1, h)
    v2d = values.reshape(-1, h)
    cb_bf16 = codebook.astype(jnp.bfloat16)
    c2 = jnp.sum(codebook * codebook, axis=1)[None, :]
    kx2 = jnp.sum(k2d * k2d, axis=1, keepdims=True)
    vx2 = jnp.sum(v2d * v2d, axis=1, keepdims=True)

    key_idx = _argmin_indices(k2d.astype(jnp.bfloat16), cb_bf16, c2, kx2)
    val_idx = _argmin_indices(v2d.astype(jnp.bfloat16), cb_bf16, c2, vx2)

    cb_masked = _mask_codebook(codebook)
    keys_c = _sc_gather(cb_masked, key_idx).reshape(batch, seq, h)
    vals_c = _sc_gather(cb_masked, val_idx).reshape(batch, seq, h)
    return keys_c, vals_c


# speedup vs baseline: 1.4068x; 1.2079x over previous
import functools

import jax
import jax.numpy as jnp
from jax import lax
from jax.experimental import pallas as pl
from jax.experimental.pallas import tpu as pltpu
from jax.experimental.pallas import tpu_sc as plsc

HIDDEN = 2048
NUM_CENTROIDS = 1024
SPARSITY_THRESHOLD = 0.1
BLK = 256


def _argmin_body(x_ref, cb_ref, c2_ref, x2_ref, idx_ref):
    xb = x_ref[...]
    m = jax.lax.dot_general(
        xb, cb_ref[...], (((1,), (1,)), ((), ())),
        preferred_element_type=jnp.float32,
    )
    d2 = x2_ref[...] + c2_ref[...] - 2.0 * m
    d = jnp.sqrt(jnp.maximum(d2, 0.0))
    dmin = jnp.min(d, axis=1, keepdims=True)
    ids = jax.lax.broadcasted_iota(jnp.int32, (BLK, NUM_CENTROIDS), 1)
    idx = jnp.min(jnp.where(d == dmin, ids, NUM_CENTROIDS), axis=1)
    idx_ref[...] = idx.reshape(1, 1, BLK)


@functools.partial(jax.jit, static_argnames=("interpret",))
def _argmin_indices(xb, cb_bf16, c2, x2, interpret=False):
    n, h = xb.shape
    k = cb_bf16.shape[0]
    nblk = n // BLK
    idx3 = pl.pallas_call(
        _argmin_body,
        grid=(nblk,),
        in_specs=[
            pl.BlockSpec((BLK, h), lambda i: (i, 0)),
            pl.BlockSpec((k, h), lambda i: (0, 0)),
            pl.BlockSpec((1, k), lambda i: (0, 0)),
            pl.BlockSpec((BLK, 1), lambda i: (i, 0)),
        ],
        out_specs=pl.BlockSpec((1, 1, BLK), lambda i: (i, 0, 0)),
        out_shape=jax.ShapeDtypeStruct((nblk, 1, BLK), jnp.int32),
        interpret=interpret,
    )(xb, cb_bf16, c2, x2)
    return idx3.reshape(n)


def _mask_body(cb_ref, out_ref, outb_ref):
    cb = cb_ref[...]
    c2 = jnp.sum(cb * cb, axis=1, keepdims=True)
    msk = (jnp.sqrt(c2) > SPARSITY_THRESHOLD).astype(cb.dtype)
    cbm = cb * msk
    out_ref[...] = cbm
    outb_ref[...] = cbm.astype(jnp.bfloat16)


@jax.jit
def _mask_codebook(codebook):
    k, h = codebook.shape
    return pl.pallas_call(
        _mask_body,
        in_specs=[pl.BlockSpec((k, h), lambda: (0, 0))],
        out_specs=[pl.BlockSpec((k, h), lambda: (0, 0)),
                   pl.BlockSpec((k, h), lambda: (0, 0))],
        out_shape=[jax.ShapeDtypeStruct((k, h), jnp.float32),
                   jax.ShapeDtypeStruct((k, h), jnp.bfloat16)],
    )(codebook)


def _argmin_gather_body(x_ref, cb_ref, c2_ref, x2_ref, cbm_ref, out_ref):
    xb = x_ref[...]
    m = jax.lax.dot_general(
        xb, cb_ref[...], (((1,), (1,)), ((), ())),
        preferred_element_type=jnp.float32,
    )
    d2 = x2_ref[...] + c2_ref[...] - 2.0 * m
    d = jnp.sqrt(jnp.maximum(d2, 0.0))
    dmin = jnp.min(d, axis=1, keepdims=True)
    ids = jax.lax.broadcasted_iota(jnp.int32, (BLK, NUM_CENTROIDS), 1)
    idx = jnp.min(jnp.where(d == dmin, ids, NUM_CENTROIDS), axis=1)
    oh = (ids == idx[:, None]).astype(jnp.bfloat16)
    out_ref[...] = jax.lax.dot_general(
        oh, cbm_ref[...], (((1,), (0,)), ((), ())),
        preferred_element_type=jnp.float32,
    )


@jax.jit
def _argmin_gather(xb, cb_bf16, c2, x2, cbm_bf16):
    n, h = xb.shape
    k = cb_bf16.shape[0]
    nblk = n // BLK
    return pl.pallas_call(
        _argmin_gather_body,
        grid=(nblk,),
        in_specs=[
            pl.BlockSpec((BLK, h), lambda i: (i, 0)),
            pl.BlockSpec((k, h), lambda i: (0, 0)),
            pl.BlockSpec((1, k), lambda i: (0, 0)),
            pl.BlockSpec((BLK, 1), lambda i: (i, 0)),
            pl.BlockSpec((k, h), lambda i: (0, 0)),
        ],
        out_specs=pl.BlockSpec((BLK, h), lambda i: (i, 0)),
        out_shape=jax.ShapeDtypeStruct((n, h), jnp.float32),
    )(xb, cb_bf16, c2, x2, cbm_bf16)


_SC_INFO = plsc.get_sparse_core_info()
_NC = _SC_INFO.num_cores
_NS = _SC_INFO.num_subcores
_NW = _NC * _NS
N_TOK = 16384
B_PER_W = N_TOK // _NW
ROWS = 8
NCHUNK = B_PER_W // ROWS
NBUF = 4


@functools.partial(
    pl.kernel,
    mesh=plsc.VectorSubcoreMesh(core_axis_name="c", subcore_axis_name="s"),
    out_type=jax.ShapeDtypeStruct((N_TOK, HIDDEN), jnp.float32),
    scratch_types=[
        pltpu.VMEM((B_PER_W,), jnp.int32),
        pltpu.VMEM((ROWS, HIDDEN), jnp.float32),
        pltpu.VMEM((ROWS, HIDDEN), jnp.float32),
        pltpu.VMEM((ROWS, HIDDEN), jnp.float32),
        pltpu.VMEM((ROWS, HIDDEN), jnp.float32),
        pltpu.SemaphoreType.DMA,
        pltpu.SemaphoreType.DMA,
        pltpu.SemaphoreType.DMA,
        pltpu.SemaphoreType.DMA,
        pltpu.SemaphoreType.DMA,
        pltpu.SemaphoreType.DMA,
        pltpu.SemaphoreType.DMA,
        pltpu.SemaphoreType.DMA,
    ],
)
def _sc_gather(table_hbm, idx_hbm, out_hbm, idx_v,
               buf0, buf1, buf2, buf3,
               gs0, gs1, gs2, gs3, ws0, ws1, ws2, ws3):
    wid = lax.axis_index("s") * _NC + lax.axis_index("c")
    base = wid * B_PER_W
    pltpu.sync_copy(idx_hbm.at[pl.ds(base, B_PER_W)], idx_v)
    bufs = (buf0, buf1, buf2, buf3)
    gsems = (gs0, gs1, gs2, gs3)
    wsems = (ws0, ws1, ws2, ws3)

    def start_gather(c, b):
        pltpu.async_copy(
            table_hbm.at[idx_v.at[pl.ds(c * ROWS, ROWS)]], bufs[b], gsems[b])

    def wait_gather(c, b):
        pltpu.make_async_copy(
            table_hbm.at[idx_v.at[pl.ds(c * ROWS, ROWS)]],
            bufs[b], gsems[b]).wait()

    def start_write(c, b):
        pltpu.async_copy(
            bufs[b], out_hbm.at[pl.ds(base + c * ROWS, ROWS)], wsems[b])

    def wait_write(c, b):
        pltpu.make_async_copy(
            bufs[b], out_hbm.at[pl.ds(base + c * ROWS, ROWS)], wsems[b]).wait()

    start_gather(0, 0)
    start_gather(1, 1)

    def slot(c, b):
        wait_gather(c, b)
        start_write(c, b)
        bp = (b + 2) % NBUF

        @pl.when(c + 2 < NCHUNK)
        def _():
            @pl.when(c + 2 >= NBUF)
            def _():
                wait_write(c + 2 - NBUF, bp)
            start_gather(c + 2, bp)

    def outer(i, _):
        for b in range(NBUF):
            slot(i * NBUF + b, b)
        return ()

    lax.fori_loop(0, NCHUNK // NBUF, outer, (), unroll=False)
    for w in range(NCHUNK - NBUF, NCHUNK):
        wait_write(w, w % NBUF)


def kernel(keys, values, codebook):
    batch, seq, h = keys.shape
    k2d = keys.reshape(-1, h)
    v2d = values.reshape(-1, h)
    cb_bf16 = codebook.astype(jnp.bfloat16)
    c2 = jnp.sum(codebook * codebook, axis=1)[None, :]
    kx2 = jnp.sum(k2d * k2d, axis=1, keepdims=True)
    vx2 = jnp.sum(v2d * v2d, axis=1, keepdims=True)

    cb_masked, cbm_bf16 = _mask_codebook(codebook)

    key_idx = _argmin_indices(k2d.astype(jnp.bfloat16), cb_bf16, c2, kx2)
    keys_c = _sc_gather(cb_masked, key_idx).reshape(batch, seq, h)
    vals_c = _argmin_gather(
        v2d.astype(jnp.bfloat16), cb_bf16, c2, vx2, cbm_bf16,
    ).reshape(batch, seq, h)
    return keys_c, vals_c


# speedup vs baseline: 1.6178x; 1.1500x over previous
import functools

import jax
import jax.numpy as jnp
from jax import lax
from jax.experimental import pallas as pl
from jax.experimental.pallas import tpu as pltpu
from jax.experimental.pallas import tpu_sc as plsc

HIDDEN = 2048
NUM_CENTROIDS = 1024
SPARSITY_THRESHOLD = 0.1
BLK = 256


def _argmin_body(x_ref, cb_ref, c2_ref, x2_ref, idx_ref):
    xb = x_ref[...]
    m = jax.lax.dot_general(
        xb, cb_ref[...], (((1,), (1,)), ((), ())),
        preferred_element_type=jnp.float32,
    )
    d2 = x2_ref[...] + c2_ref[...] - 2.0 * m
    d = jnp.sqrt(jnp.maximum(d2, 0.0))
    dmin = jnp.min(d, axis=1, keepdims=True)
    ids = jax.lax.broadcasted_iota(jnp.int32, (BLK, NUM_CENTROIDS), 1)
    idx = jnp.min(jnp.where(d == dmin, ids, NUM_CENTROIDS), axis=1)
    idx_ref[...] = idx.reshape(1, 1, BLK)


@functools.partial(jax.jit, static_argnames=("interpret",))
def _argmin_indices(xb, cb_bf16, c2, x2, interpret=False):
    n, h = xb.shape
    k = cb_bf16.shape[0]
    nblk = n // BLK
    idx3 = pl.pallas_call(
        _argmin_body,
        grid=(nblk,),
        in_specs=[
            pl.BlockSpec((BLK, h), lambda i: (i, 0)),
            pl.BlockSpec((k, h), lambda i: (0, 0)),
            pl.BlockSpec((1, k), lambda i: (0, 0)),
            pl.BlockSpec((BLK, 1), lambda i: (i, 0)),
        ],
        out_specs=pl.BlockSpec((1, 1, BLK), lambda i: (i, 0, 0)),
        out_shape=jax.ShapeDtypeStruct((nblk, 1, BLK), jnp.int32),
        interpret=interpret,
    )(xb, cb_bf16, c2, x2)
    return idx3.reshape(n)


def _mask_body(cb_ref, out_ref, outb_ref):
    cb = cb_ref[...]
    c2 = jnp.sum(cb * cb, axis=1, keepdims=True)
    msk = (jnp.sqrt(c2) > SPARSITY_THRESHOLD).astype(cb.dtype)
    cbm = cb * msk
    out_ref[...] = cbm
    outb_ref[...] = cbm.astype(jnp.bfloat16)


@jax.jit
def _mask_codebook(codebook):
    k, h = codebook.shape
    return pl.pallas_call(
        _mask_body,
        in_specs=[pl.BlockSpec((k, h), lambda: (0, 0))],
        out_specs=[pl.BlockSpec((k, h), lambda: (0, 0)),
                   pl.BlockSpec((k, h), lambda: (0, 0))],
        out_shape=[jax.ShapeDtypeStruct((k, h), jnp.float32),
                   jax.ShapeDtypeStruct((k, h), jnp.bfloat16)],
    )(codebook)


def _argmin_gather_body(x_ref, cb_ref, c2_ref, x2_ref, cbm_ref, out_ref):
    xb = x_ref[...]
    m = jax.lax.dot_general(
        xb, cb_ref[...], (((1,), (1,)), ((), ())),
        preferred_element_type=jnp.float32,
    )
    d2 = x2_ref[...] + c2_ref[...] - 2.0 * m
    d = jnp.sqrt(jnp.maximum(d2, 0.0))
    dmin = jnp.min(d, axis=1, keepdims=True)
    ids = jax.lax.broadcasted_iota(jnp.int32, (BLK, NUM_CENTROIDS), 1)
    idx = jnp.min(jnp.where(d == dmin, ids, NUM_CENTROIDS), axis=1)
    oh = (ids == idx[:, None]).astype(jnp.bfloat16)
    out_ref[...] = jax.lax.dot_general(
        oh, cbm_ref[...], (((1,), (0,)), ((), ())),
        preferred_element_type=jnp.float32,
    )


@jax.jit
def _argmin_gather(xb, cb_bf16, c2, x2, cbm_bf16):
    n, h = xb.shape
    k = cb_bf16.shape[0]
    nblk = n // BLK
    return pl.pallas_call(
        _argmin_gather_body,
        grid=(nblk,),
        in_specs=[
            pl.BlockSpec((BLK, h), lambda i: (i, 0)),
            pl.BlockSpec((k, h), lambda i: (0, 0)),
            pl.BlockSpec((1, k), lambda i: (0, 0)),
            pl.BlockSpec((BLK, 1), lambda i: (i, 0)),
            pl.BlockSpec((k, h), lambda i: (0, 0)),
        ],
        out_specs=pl.BlockSpec((BLK, h), lambda i: (i, 0)),
        out_shape=jax.ShapeDtypeStruct((n, h), jnp.float32),
    )(xb, cb_bf16, c2, x2, cbm_bf16)


_SC_INFO = plsc.get_sparse_core_info()
_NC = _SC_INFO.num_cores
_NS = _SC_INFO.num_subcores
_NW = _NC * _NS
N_TOK = 16384
B_PER_W = N_TOK // _NW
ROWS = 8
NCHUNK = B_PER_W // ROWS
NBUF = 4


@functools.partial(
    pl.kernel,
    mesh=plsc.VectorSubcoreMesh(core_axis_name="c", subcore_axis_name="s"),
    out_type=jax.ShapeDtypeStruct((N_TOK, HIDDEN), jnp.float32),
    scratch_types=[
        pltpu.VMEM((B_PER_W,), jnp.int32),
        pltpu.VMEM((ROWS, HIDDEN), jnp.float32),
        pltpu.VMEM((ROWS, HIDDEN), jnp.float32),
        pltpu.VMEM((ROWS, HIDDEN), jnp.float32),
        pltpu.VMEM((ROWS, HIDDEN), jnp.float32),
        pltpu.SemaphoreType.DMA,
        pltpu.SemaphoreType.DMA,
        pltpu.SemaphoreType.DMA,
        pltpu.SemaphoreType.DMA,
        pltpu.SemaphoreType.DMA,
        pltpu.SemaphoreType.DMA,
        pltpu.SemaphoreType.DMA,
        pltpu.SemaphoreType.DMA,
    ],
)
def _sc_gather(table_hbm, idx_hbm, out_hbm, idx_v,
               buf0, buf1, buf2, buf3,
               gs0, gs1, gs2, gs3, ws0, ws1, ws2, ws3):
    wid = lax.axis_index("s") * _NC + lax.axis_index("c")
    base = wid * B_PER_W
    pltpu.sync_copy(idx_hbm.at[pl.ds(base, B_PER_W)], idx_v)
    bufs = (buf0, buf1, buf2, buf3)
    gsems = (gs0, gs1, gs2, gs3)
    wsems = (ws0, ws1, ws2, ws3)

    def start_gather(c, b):
        pltpu.async_copy(
            table_hbm.at[idx_v.at[pl.ds(c * ROWS, ROWS)]], bufs[b], gsems[b])

    def wait_gather(c, b):
        pltpu.make_async_copy(
            table_hbm.at[idx_v.at[pl.ds(c * ROWS, ROWS)]],
            bufs[b], gsems[b]).wait()

    def start_write(c, b):
        pltpu.async_copy(
            bufs[b], out_hbm.at[pl.ds(base + c * ROWS, ROWS)], wsems[b])

    def wait_write(c, b):
        pltpu.make_async_copy(
            bufs[b], out_hbm.at[pl.ds(base + c * ROWS, ROWS)], wsems[b]).wait()

    start_gather(0, 0)
    start_gather(1, 1)

    def slot(c, b):
        wait_gather(c, b)
        start_write(c, b)
        bp = (b + 2) % NBUF

        @pl.when(c + 2 < NCHUNK)
        def _():
            @pl.when(c + 2 >= NBUF)
            def _():
                wait_write(c + 2 - NBUF, bp)
            start_gather(c + 2, bp)

    def outer(i, _):
        for b in range(NBUF):
            slot(i * NBUF + b, b)
        return ()

    lax.fori_loop(0, NCHUNK // NBUF, outer, (), unroll=False)
    for w in range(NCHUNK - NBUF, NCHUNK):
        wait_write(w, w % NBUF)


def kernel(keys, values, codebook):
    batch, seq, h = keys.shape
    k2d = keys.reshape(-1, h)
    v2d = values.reshape(-1, h)
    cb_bf16 = codebook.astype(jnp.bfloat16)
    c2 = jnp.sum(codebook * codebook, axis=1)[None, :]
    kx2 = jnp.sum(k2d * k2d, axis=1, keepdims=True)
    vx2 = jnp.sum(v2d * v2d, axis=1, keepdims=True)

    cb_masked, cbm_bf16 = _mask_codebook(codebook)

    key_idx = _argmin_indices(k2d.astype(jnp.bfloat16), cb_bf16, c2, kx2)
    vb, vx2b, key_idx_b = jax.lax.optimization_barrier(
        (v2d.astype(jnp.bfloat16), vx2, key_idx))
    keys_c = _sc_gather(cb_masked, key_idx_b).reshape(batch, seq, h)
    vals_c = _argmin_gather(vb, cb_bf16, c2, vx2b, cbm_bf16).reshape(batch, seq, h)
    return keys_c, vals_c


# speedup vs baseline: 1.6849x; 1.0415x over previous
import functools

import jax
import jax.numpy as jnp
from jax import lax
from jax.experimental import pallas as pl
from jax.experimental.pallas import tpu as pltpu
from jax.experimental.pallas import tpu_sc as plsc

HIDDEN = 2048
NUM_CENTROIDS = 1024
SPARSITY_THRESHOLD = 0.1
BLK = 512


def _argmin_body(x_ref, cb_ref, c2_ref, x2_ref, idx_ref):
    xb = x_ref[...]
    m = jax.lax.dot_general(
        xb, cb_ref[...], (((1,), (1,)), ((), ())),
        preferred_element_type=jnp.float32,
    )
    d2 = x2_ref[...] + c2_ref[...] - 2.0 * m
    d = jnp.sqrt(jnp.maximum(d2, 0.0))
    dmin = jnp.min(d, axis=1, keepdims=True)
    ids = jax.lax.broadcasted_iota(jnp.int32, (BLK, NUM_CENTROIDS), 1)
    idx = jnp.min(jnp.where(d == dmin, ids, NUM_CENTROIDS), axis=1)
    idx_ref[...] = idx.reshape(1, 1, BLK)


@functools.partial(jax.jit, static_argnames=("interpret",))
def _argmin_indices(xb, cb_bf16, c2, x2, interpret=False):
    n, h = xb.shape
    k = cb_bf16.shape[0]
    nblk = n // BLK
    idx3 = pl.pallas_call(
        _argmin_body,
        grid=(nblk,),
        in_specs=[
            pl.BlockSpec((BLK, h), lambda i: (i, 0)),
            pl.BlockSpec((k, h), lambda i: (0, 0)),
            pl.BlockSpec((1, k), lambda i: (0, 0)),
            pl.BlockSpec((BLK, 1), lambda i: (i, 0)),
        ],
        out_specs=pl.BlockSpec((1, 1, BLK), lambda i: (i, 0, 0)),
        out_shape=jax.ShapeDtypeStruct((nblk, 1, BLK), jnp.int32),
        interpret=interpret,
    )(xb, cb_bf16, c2, x2)
    return idx3.reshape(n)


def _mask_body(cb_ref, out_ref, outb_ref):
    cb = cb_ref[...]
    c2 = jnp.sum(cb * cb, axis=1, keepdims=True)
    msk = (jnp.sqrt(c2) > SPARSITY_THRESHOLD).astype(cb.dtype)
    cbm = cb * msk
    out_ref[...] = cbm
    outb_ref[...] = cbm.astype(jnp.bfloat16)


@jax.jit
def _mask_codebook(codebook):
    k, h = codebook.shape
    return pl.pallas_call(
        _mask_body,
        in_specs=[pl.BlockSpec((k, h), lambda: (0, 0))],
        out_specs=[pl.BlockSpec((k, h), lambda: (0, 0)),
                   pl.BlockSpec((k, h), lambda: (0, 0))],
        out_shape=[jax.ShapeDtypeStruct((k, h), jnp.float32),
                   jax.ShapeDtypeStruct((k, h), jnp.bfloat16)],
    )(codebook)


def _argmin_gather_body(x_ref, cb_ref, c2_ref, x2_ref, cbm_ref, out_ref):
    xb = x_ref[...]
    m = jax.lax.dot_general(
        xb, cb_ref[...], (((1,), (1,)), ((), ())),
        preferred_element_type=jnp.float32,
    )
    d2 = x2_ref[...] + c2_ref[...] - 2.0 * m
    d = jnp.sqrt(jnp.maximum(d2, 0.0))
    dmin = jnp.min(d, axis=1, keepdims=True)
    ids = jax.lax.broadcasted_iota(jnp.int32, (BLK, NUM_CENTROIDS), 1)
    idx = jnp.min(jnp.where(d == dmin, ids, NUM_CENTROIDS), axis=1)
    oh = (ids == idx[:, None]).astype(jnp.bfloat16)
    out_ref[...] = jax.lax.dot_general(
        oh, cbm_ref[...], (((1,), (0,)), ((), ())),
        preferred_element_type=jnp.float32,
    )


@jax.jit
def _argmin_gather(xb, cb_bf16, c2, x2, cbm_bf16):
    n, h = xb.shape
    k = cb_bf16.shape[0]
    nblk = n // BLK
    return pl.pallas_call(
        _argmin_gather_body,
        grid=(nblk,),
        in_specs=[
            pl.BlockSpec((BLK, h), lambda i: (i, 0)),
            pl.BlockSpec((k, h), lambda i: (0, 0)),
            pl.BlockSpec((1, k), lambda i: (0, 0)),
            pl.BlockSpec((BLK, 1), lambda i: (i, 0)),
            pl.BlockSpec((k, h), lambda i: (0, 0)),
        ],
        out_specs=pl.BlockSpec((BLK, h), lambda i: (i, 0)),
        out_shape=jax.ShapeDtypeStruct((n, h), jnp.float32),
    )(xb, cb_bf16, c2, x2, cbm_bf16)


_SC_INFO = plsc.get_sparse_core_info()
_NC = _SC_INFO.num_cores
_NS = _SC_INFO.num_subcores
_NW = _NC * _NS
N_TOK = 16384
B_PER_W = N_TOK // _NW
ROWS = 8
NCHUNK = B_PER_W // ROWS
NBUF = 4


@functools.partial(
    pl.kernel,
    mesh=plsc.VectorSubcoreMesh(core_axis_name="c", subcore_axis_name="s"),
    out_type=jax.ShapeDtypeStruct((N_TOK, HIDDEN), jnp.float32),
    scratch_types=[
        pltpu.VMEM((B_PER_W,), jnp.int32),
        pltpu.VMEM((ROWS, HIDDEN), jnp.float32),
        pltpu.VMEM((ROWS, HIDDEN), jnp.float32),
        pltpu.VMEM((ROWS, HIDDEN), jnp.float32),
        pltpu.VMEM((ROWS, HIDDEN), jnp.float32),
        pltpu.SemaphoreType.DMA,
        pltpu.SemaphoreType.DMA,
        pltpu.SemaphoreType.DMA,
        pltpu.SemaphoreType.DMA,
        pltpu.SemaphoreType.DMA,
        pltpu.SemaphoreType.DMA,
        pltpu.SemaphoreType.DMA,
        pltpu.SemaphoreType.DMA,
    ],
)
def _sc_gather(table_hbm, idx_hbm, out_hbm, idx_v,
               buf0, buf1, buf2, buf3,
               gs0, gs1, gs2, gs3, ws0, ws1, ws2, ws3):
    wid = lax.axis_index("s") * _NC + lax.axis_index("c")
    base = wid * B_PER_W
    pltpu.sync_copy(idx_hbm.at[pl.ds(base, B_PER_W)], idx_v)
    bufs = (buf0, buf1, buf2, buf3)
    gsems = (gs0, gs1, gs2, gs3)
    wsems = (ws0, ws1, ws2, ws3)

    def start_gather(c, b):
        pltpu.async_copy(
            table_hbm.at[idx_v.at[pl.ds(c * ROWS, ROWS)]], bufs[b], gsems[b])

    def wait_gather(c, b):
        pltpu.make_async_copy(
            table_hbm.at[idx_v.at[pl.ds(c * ROWS, ROWS)]],
            bufs[b], gsems[b]).wait()

    def start_write(c, b):
        pltpu.async_copy(
            bufs[b], out_hbm.at[pl.ds(base + c * ROWS, ROWS)], wsems[b])

    def wait_write(c, b):
        pltpu.make_async_copy(
            bufs[b], out_hbm.at[pl.ds(base + c * ROWS, ROWS)], wsems[b]).wait()

    start_gather(0, 0)
    start_gather(1, 1)

    def slot(c, b):
        wait_gather(c, b)
        start_write(c, b)
        bp = (b + 2) % NBUF

        @pl.when(c + 2 < NCHUNK)
        def _():
            @pl.when(c + 2 >= NBUF)
            def _():
                wait_write(c + 2 - NBUF, bp)
            start_gather(c + 2, bp)

    def outer(i, _):
        for b in range(NBUF):
            slot(i * NBUF + b, b)
        return ()

    lax.fori_loop(0, NCHUNK // NBUF, outer, (), unroll=False)
    for w in range(NCHUNK - NBUF, NCHUNK):
        wait_write(w, w % NBUF)


def kernel(keys, values, codebook):
    batch, seq, h = keys.shape
    k2d = keys.reshape(-1, h)
    v2d = values.reshape(-1, h)
    cb_bf16 = codebook.astype(jnp.bfloat16)
    c2 = jnp.sum(codebook * codebook, axis=1)[None, :]
    kx2 = jnp.sum(k2d * k2d, axis=1, keepdims=True)
    vx2 = jnp.sum(v2d * v2d, axis=1, keepdims=True)

    cb_masked, cbm_bf16 = _mask_codebook(codebook)

    key_idx = _argmin_indices(k2d.astype(jnp.bfloat16), cb_bf16, c2, kx2)
    vb, vx2b, key_idx_b = jax.lax.optimization_barrier(
        (v2d.astype(jnp.bfloat16), vx2, key_idx))
    keys_c = _sc_gather(cb_masked, key_idx_b).reshape(batch, seq, h)
    vals_c = _argmin_gather(vb, cb_bf16, c2, vx2b, cbm_bf16).reshape(batch, seq, h)
    return keys_c, vals_c


# speedup vs baseline: 1.6938x; 1.0052x over previous
import functools

import jax
import jax.numpy as jnp
from jax import lax
from jax.experimental import pallas as pl
from jax.experimental.pallas import tpu as pltpu
from jax.experimental.pallas import tpu_sc as plsc

HIDDEN = 2048
NUM_CENTROIDS = 1024
SPARSITY_THRESHOLD = 0.1
BLK = 1024


def _argmin_body(x_ref, cb_ref, c2_ref, x2_ref, idx_ref):
    xb = x_ref[...]
    m = jax.lax.dot_general(
        xb, cb_ref[...], (((1,), (1,)), ((), ())),
        preferred_element_type=jnp.float32,
    )
    d2 = x2_ref[...] + c2_ref[...] - 2.0 * m
    d = jnp.sqrt(jnp.maximum(d2, 0.0))
    dmin = jnp.min(d, axis=1, keepdims=True)
    ids = jax.lax.broadcasted_iota(jnp.int32, (BLK, NUM_CENTROIDS), 1)
    idx = jnp.min(jnp.where(d == dmin, ids, NUM_CENTROIDS), axis=1)
    idx_ref[...] = idx.reshape(1, 1, BLK)


@functools.partial(jax.jit, static_argnames=("interpret",))
def _argmin_indices(xb, cb_bf16, c2, x2, interpret=False):
    n, h = xb.shape
    k = cb_bf16.shape[0]
    nblk = n // BLK
    idx3 = pl.pallas_call(
        _argmin_body,
        grid=(nblk,),
        in_specs=[
            pl.BlockSpec((BLK, h), lambda i: (i, 0)),
            pl.BlockSpec((k, h), lambda i: (0, 0)),
            pl.BlockSpec((1, k), lambda i: (0, 0)),
            pl.BlockSpec((BLK, 1), lambda i: (i, 0)),
        ],
        out_specs=pl.BlockSpec((1, 1, BLK), lambda i: (i, 0, 0)),
        out_shape=jax.ShapeDtypeStruct((nblk, 1, BLK), jnp.int32),
        interpret=interpret,
    )(xb, cb_bf16, c2, x2)
    return idx3.reshape(n)


def _mask_body(cb_ref, out_ref, outb_ref):
    cb = cb_ref[...]
    c2 = jnp.sum(cb * cb, axis=1, keepdims=True)
    msk = (jnp.sqrt(c2) > SPARSITY_THRESHOLD).astype(cb.dtype)
    cbm = cb * msk
    out_ref[...] = cbm
    outb_ref[...] = cbm.astype(jnp.bfloat16)


@jax.jit
def _mask_codebook(codebook):
    k, h = codebook.shape
    return pl.pallas_call(
        _mask_body,
        in_specs=[pl.BlockSpec((k, h), lambda: (0, 0))],
        out_specs=[pl.BlockSpec((k, h), lambda: (0, 0)),
                   pl.BlockSpec((k, h), lambda: (0, 0))],
        out_shape=[jax.ShapeDtypeStruct((k, h), jnp.float32),
                   jax.ShapeDtypeStruct((k, h), jnp.bfloat16)],
    )(codebook)


def _argmin_gather_body(x_ref, cb_ref, c2_ref, x2_ref, cbm_ref, out_ref):
    xb = x_ref[...]
    m = jax.lax.dot_general(
        xb, cb_ref[...], (((1,), (1,)), ((), ())),
        preferred_element_type=jnp.float32,
    )
    d2 = x2_ref[...] + c2_ref[...] - 2.0 * m
    d = jnp.sqrt(jnp.maximum(d2, 0.0))
    dmin = jnp.min(d, axis=1, keepdims=True)
    ids = jax.lax.broadcasted_iota(jnp.int32, (BLK, NUM_CENTROIDS), 1)
    idx = jnp.min(jnp.where(d == dmin, ids, NUM_CENTROIDS), axis=1)
    oh = (ids == idx[:, None]).astype(jnp.bfloat16)
    out_ref[...] = jax.lax.dot_general(
        oh, cbm_ref[...], (((1,), (0,)), ((), ())),
        preferred_element_type=jnp.float32,
    )


@jax.jit
def _argmin_gather(xb, cb_bf16, c2, x2, cbm_bf16):
    n, h = xb.shape
    k = cb_bf16.shape[0]
    nblk = n // BLK
    return pl.pallas_call(
        _argmin_gather_body,
        grid=(nblk,),
        in_specs=[
            pl.BlockSpec((BLK, h), lambda i: (i, 0)),
            pl.BlockSpec((k, h), lambda i: (0, 0)),
            pl.BlockSpec((1, k), lambda i: (0, 0)),
            pl.BlockSpec((BLK, 1), lambda i: (i, 0)),
            pl.BlockSpec((k, h), lambda i: (0, 0)),
        ],
        out_specs=pl.BlockSpec((BLK, h), lambda i: (i, 0)),
        out_shape=jax.ShapeDtypeStruct((n, h), jnp.float32),
    )(xb, cb_bf16, c2, x2, cbm_bf16)


_SC_INFO = plsc.get_sparse_core_info()
_NC = _SC_INFO.num_cores
_NS = _SC_INFO.num_subcores
_NW = _NC * _NS
N_TOK = 16384
B_PER_W = N_TOK // _NW
ROWS = 8
NCHUNK = B_PER_W // ROWS
NBUF = 4


@functools.partial(
    pl.kernel,
    mesh=plsc.VectorSubcoreMesh(core_axis_name="c", subcore_axis_name="s"),
    out_type=jax.ShapeDtypeStruct((N_TOK, HIDDEN), jnp.float32),
    scratch_types=[
        pltpu.VMEM((B_PER_W,), jnp.int32),
        pltpu.VMEM((ROWS, HIDDEN), jnp.float32),
        pltpu.VMEM((ROWS, HIDDEN), jnp.float32),
        pltpu.VMEM((ROWS, HIDDEN), jnp.float32),
        pltpu.VMEM((ROWS, HIDDEN), jnp.float32),
        pltpu.SemaphoreType.DMA,
        pltpu.SemaphoreType.DMA,
        pltpu.SemaphoreType.DMA,
        pltpu.SemaphoreType.DMA,
        pltpu.SemaphoreType.DMA,
        pltpu.SemaphoreType.DMA,
        pltpu.SemaphoreType.DMA,
        pltpu.SemaphoreType.DMA,
    ],
)
def _sc_gather(table_hbm, idx_hbm, out_hbm, idx_v,
               buf0, buf1, buf2, buf3,
               gs0, gs1, gs2, gs3, ws0, ws1, ws2, ws3):
    wid = lax.axis_index("s") * _NC + lax.axis_index("c")
    base = wid * B_PER_W
    pltpu.sync_copy(idx_hbm.at[pl.ds(base, B_PER_W)], idx_v)
    bufs = (buf0, buf1, buf2, buf3)
    gsems = (gs0, gs1, gs2, gs3)
    wsems = (ws0, ws1, ws2, ws3)

    def start_gather(c, b):
        pltpu.async_copy(
            table_hbm.at[idx_v.at[pl.ds(c * ROWS, ROWS)]], bufs[b], gsems[b])

    def wait_gather(c, b):
        pltpu.make_async_copy(
            table_hbm.at[idx_v.at[pl.ds(c * ROWS, ROWS)]],
            bufs[b], gsems[b]).wait()

    def start_write(c, b):
        pltpu.async_copy(
            bufs[b], out_hbm.at[pl.ds(base + c * ROWS, ROWS)], wsems[b])

    def wait_write(c, b):
        pltpu.make_async_copy(
            bufs[b], out_hbm.at[pl.ds(base + c * ROWS, ROWS)], wsems[b]).wait()

    start_gather(0, 0)
    start_gather(1, 1)

    def slot(c, b):
        wait_gather(c, b)
        start_write(c, b)
        bp = (b + 2) % NBUF

        @pl.when(c + 2 < NCHUNK)
        def _():
            @pl.when(c + 2 >= NBUF)
            def _():
                wait_write(c + 2 - NBUF, bp)
            start_gather(c + 2, bp)

    def outer(i, _):
        for b in range(NBUF):
            slot(i * NBUF + b, b)
        return ()

    lax.fori_loop(0, NCHUNK // NBUF, outer, (), unroll=False)
    for w in range(NCHUNK - NBUF, NCHUNK):
        wait_write(w, w % NBUF)


def kernel(keys, values, codebook):
    batch, seq, h = keys.shape
    k2d = keys.reshape(-1, h)
    v2d = values.reshape(-1, h)
    cb_bf16 = codebook.astype(jnp.bfloat16)
    c2 = jnp.sum(codebook * codebook, axis=1)[None, :]
    kx2 = jnp.sum(k2d * k2d, axis=1, keepdims=True)
    vx2 = jnp.sum(v2d * v2d, axis=1, keepdims=True)

    cb_masked, cbm_bf16 = _mask_codebook(codebook)

    key_idx = _argmin_indices(k2d.astype(jnp.bfloat16), cb_bf16, c2, kx2)
    vb, vx2b, key_idx_b = jax.lax.optimization_barrier(
        (v2d.astype(jnp.bfloat16), vx2, key_idx))
    keys_c = _sc_gather(cb_masked, key_idx_b).reshape(batch, seq, h)
    vals_c = _argmin_gather(vb, cb_bf16, c2, vx2b, cbm_bf16).reshape(batch, seq, h)
    return keys_c, vals_c
